# baseline (speedup 1.0000x reference)
import functools

import jax
import jax.numpy as jnp
import numpy as np
from jax import lax
from jax.experimental import pallas as pl
from jax.experimental.pallas import tpu as pltpu
from jax.experimental.pallas import tpu_sc as plsc

_PROB = 0.12
_T = 4096
_ROWS = 16 * 256
_NW = 32
_ROWS_PER_TILE = _ROWS // _NW
_RB = 8
_BUFW = _RB * _T
_NBLK = _ROWS_PER_TILE // _RB
_L = 16


def _rotl(x, d):
    return ((x << np.uint32(d)) | (x >> np.uint32(32 - d))).astype(np.uint32)


def _threefry2x32_core(ks0, ks1, x0, x1):
    ks2 = np.uint32(ks0 ^ ks1 ^ np.uint32(0x1BD11BDA))
    rot = [[13, 15, 26, 6], [17, 29, 16, 24]]
    x0 = (x0 + ks0).astype(np.uint32)
    x1 = (x1 + ks1).astype(np.uint32)
    inject = [(ks1, ks2), (ks2, ks0), (ks0, ks1), (ks1, ks2), (ks2, ks0)]
    for g in range(5):
        for r in rot[g % 2]:
            x0 = (x0 + x1).astype(np.uint32)
            x1 = _rotl(x1, r)
            x1 = (x1 ^ x0).astype(np.uint32)
        a, b = inject[g]
        x0 = (x0 + a).astype(np.uint32)
        x1 = (x1 + b + np.uint32(g + 1)).astype(np.uint32)
    return x0, x1


def _uniform01(ks0, ks1, n):
    b1, b2 = _threefry2x32_core(
        ks0, ks1, np.zeros(n, dtype=np.uint32), np.arange(n, dtype=np.uint32))
    bits = (b1 ^ b2).astype(np.uint32)
    return ((bits >> np.uint32(9)) | np.uint32(0x3F800000)).view(np.float32) - np.float32(1.0)


def _jitter_index_constants():
    b1, b2 = _threefry2x32_core(np.uint32(0), np.uint32(42),
                                np.zeros(2, dtype=np.uint32),
                                np.arange(2, dtype=np.uint32))
    replace = _uniform01(b1[0], b2[0], _T) < np.float32(_PROB)
    direction = np.where(_uniform01(b1[1], b2[1], _T) < np.float32(0.5), -1, 1)
    i = np.arange(_T)
    offset = np.where(i == 0, 1, np.where(i == _T - 1, -1, direction))
    final = np.where(replace, i + offset, i)
    pos = np.nonzero(final != i)[0]
    nb = final[pos]
    safe = int(np.nonzero(final == i)[0][0])
    npad = (-len(pos)) % _L
    pos = np.concatenate([pos, np.full(npad, safe)]).astype(np.int32)
    nb = np.concatenate([nb, np.full(npad, safe)]).astype(np.int32)
    return pos, nb


_POS, _NB = _jitter_index_constants()
_NFIX = len(_POS)
_NCH = _NFIX // _L


def _sc_jitter(x_flat, pos, nb):
    mesh = plsc.VectorSubcoreMesh(core_axis_name="c", subcore_axis_name="s")

    @functools.partial(
        pl.kernel,
        mesh=mesh,
        out_type=jax.ShapeDtypeStruct((_ROWS * _T,), jnp.float32),
        compiler_params=pltpu.CompilerParams(needs_layout_passes=False),
        scratch_types=[
            pltpu.VMEM((_NFIX,), jnp.int32),
            pltpu.VMEM((_NFIX,), jnp.int32),
            pltpu.VMEM((_NFIX,), jnp.float32),
            pltpu.VMEM((_BUFW,), jnp.float32),
        ],
    )
    def k(x_hbm, pos_hbm, nb_hbm, out_hbm, pos_v, nb_v, gat_v, buf_v):
        wid = lax.axis_index("s") * 2 + lax.axis_index("c")
        pltpu.sync_copy(pos_hbm, pos_v)
        pltpu.sync_copy(nb_hbm, nb_v)
        tile_base = wid * (_ROWS_PER_TILE * _T)

        def block_body(b, carry):
            base = tile_base + b * _BUFW
            pltpu.sync_copy(x_hbm.at[pl.ds(base, _BUFW)], buf_v)

            def row_body(r, c2):
                rbase = r * _T

                def gather_body(j, c3):
                    idx = nb_v[pl.ds(j * _L, _L)] + rbase
                    gat_v[pl.ds(j * _L, _L)] = plsc.load_gather(buf_v, [idx])
                    return c3

                lax.fori_loop(0, _NCH, gather_body, 0, unroll=4)

                def scatter_body(j, c3):
                    idx = pos_v[pl.ds(j * _L, _L)] + rbase
                    plsc.store_scatter(buf_v, [idx], gat_v[pl.ds(j * _L, _L)])
                    return c3

                lax.fori_loop(0, _NCH, scatter_body, 0, unroll=4)
                return c2

            lax.fori_loop(0, _RB, row_body, 0)
            pltpu.sync_copy(buf_v, out_hbm.at[pl.ds(base, _BUFW)])
            return carry

        lax.fori_loop(0, _NBLK, block_body, 0)

    return k(x_flat, pos, nb)


def kernel(quantized):
    B, C, T = quantized.shape
    x = quantized.reshape(-1)
    out = _sc_jitter(x, jnp.asarray(_POS), jnp.asarray(_NB))
    return out.reshape(B, C, T)


# speedup vs baseline: 1.0637x; 1.0637x over previous
import functools

import jax
import jax.numpy as jnp
import numpy as np
from jax import lax
from jax.experimental import pallas as pl
from jax.experimental.pallas import tpu as pltpu
from jax.experimental.pallas import tpu_sc as plsc

_PROB = 0.12
_T = 4096
_ROWS = 16 * 256
_NW = 32
_ROWS_PER_TILE = _ROWS // _NW
_RB = 8
_BUFW = _RB * _T
_NBLK = _ROWS_PER_TILE // _RB
_L = 16


def _rotl(x, d):
    return ((x << np.uint32(d)) | (x >> np.uint32(32 - d))).astype(np.uint32)


def _threefry2x32_core(ks0, ks1, x0, x1):
    ks2 = np.uint32(ks0 ^ ks1 ^ np.uint32(0x1BD11BDA))
    rot = [[13, 15, 26, 6], [17, 29, 16, 24]]
    x0 = (x0 + ks0).astype(np.uint32)
    x1 = (x1 + ks1).astype(np.uint32)
    inject = [(ks1, ks2), (ks2, ks0), (ks0, ks1), (ks1, ks2), (ks2, ks0)]
    for g in range(5):
        for r in rot[g % 2]:
            x0 = (x0 + x1).astype(np.uint32)
            x1 = _rotl(x1, r)
            x1 = (x1 ^ x0).astype(np.uint32)
        a, b = inject[g]
        x0 = (x0 + a).astype(np.uint32)
        x1 = (x1 + b + np.uint32(g + 1)).astype(np.uint32)
    return x0, x1


def _uniform01(ks0, ks1, n):
    b1, b2 = _threefry2x32_core(
        ks0, ks1, np.zeros(n, dtype=np.uint32), np.arange(n, dtype=np.uint32))
    bits = (b1 ^ b2).astype(np.uint32)
    return ((bits >> np.uint32(9)) | np.uint32(0x3F800000)).view(np.float32) - np.float32(1.0)


def _jitter_index_constants():
    b1, b2 = _threefry2x32_core(np.uint32(0), np.uint32(42),
                                np.zeros(2, dtype=np.uint32),
                                np.arange(2, dtype=np.uint32))
    replace = _uniform01(b1[0], b2[0], _T) < np.float32(_PROB)
    direction = np.where(_uniform01(b1[1], b2[1], _T) < np.float32(0.5), -1, 1)
    i = np.arange(_T)
    offset = np.where(i == 0, 1, np.where(i == _T - 1, -1, direction))
    final = np.where(replace, i + offset, i)
    pos = np.nonzero(final != i)[0]
    nb = final[pos]
    safe = int(np.nonzero(final == i)[0][0])
    npad = (-len(pos)) % _L
    pos = np.concatenate([pos, np.full(npad, safe)]).astype(np.int32)
    nb = np.concatenate([nb, np.full(npad, safe)]).astype(np.int32)
    return pos, nb


_POS, _NB = _jitter_index_constants()
_NFIX = len(_POS)
_ROFF = (np.arange(_RB, dtype=np.int32) * _T)[:, None]
_POS_BLK = (_POS[None, :] + _ROFF).ravel()
_NB_BLK = (_NB[None, :] + _ROFF).ravel()
_NFIXB = _POS_BLK.size
_NCHB = _NFIXB // _L


def _sc_jitter(x_flat, pos, nb):
    mesh = plsc.VectorSubcoreMesh(core_axis_name="c", subcore_axis_name="s")

    @functools.partial(
        pl.kernel,
        mesh=mesh,
        out_type=jax.ShapeDtypeStruct((_ROWS * _T,), jnp.float32),
        compiler_params=pltpu.CompilerParams(needs_layout_passes=False),
        scratch_types=[
            pltpu.VMEM((_NFIXB,), jnp.int32),
            pltpu.VMEM((_NFIXB,), jnp.int32),
            pltpu.VMEM((_NFIXB,), jnp.float32),
            pltpu.VMEM((_BUFW,), jnp.float32),
            pltpu.VMEM((_BUFW,), jnp.float32),
            pltpu.SemaphoreType.DMA,
            pltpu.SemaphoreType.DMA,
            pltpu.SemaphoreType.DMA,
            pltpu.SemaphoreType.DMA,
        ],
    )
    def k(x_hbm, pos_hbm, nb_hbm, out_hbm, pos_v, nb_v, gat_v,
          buf0, buf1, si0, si1, so0, so1):
        wid = lax.axis_index("s") * 2 + lax.axis_index("c")
        pltpu.sync_copy(pos_hbm, pos_v)
        pltpu.sync_copy(nb_hbm, nb_v)
        tile_base = wid * (_ROWS_PER_TILE * _T)
        bufs = (buf0, buf1)
        sins = (si0, si1)
        souts = (so0, so1)

        def fixup(buf):
            def gather_body(j, c):
                idx = nb_v[pl.ds(j * _L, _L)]
                gat_v[pl.ds(j * _L, _L)] = plsc.load_gather(buf, [idx])
                return c

            lax.fori_loop(0, _NCHB, gather_body, 0, unroll=8)

            def scatter_body(j, c):
                idx = pos_v[pl.ds(j * _L, _L)]
                plsc.store_scatter(buf, [idx], gat_v[pl.ds(j * _L, _L)])
                return c

            lax.fori_loop(0, _NCHB, scatter_body, 0, unroll=8)

        h_in = [None, None]
        h_out = [None, None]
        h_in[0] = pltpu.async_copy(
            x_hbm.at[pl.ds(tile_base, _BUFW)], bufs[0], sins[0])
        for b in range(_NBLK):
            cur = b % 2
            nxt = (b + 1) % 2
            if b + 1 < _NBLK:
                if h_out[nxt] is not None:
                    h_out[nxt].wait()
                h_in[nxt] = pltpu.async_copy(
                    x_hbm.at[pl.ds(tile_base + (b + 1) * _BUFW, _BUFW)],
                    bufs[nxt], sins[nxt])
            h_in[cur].wait()
            fixup(bufs[cur])
            h_out[cur] = pltpu.async_copy(
                bufs[cur], out_hbm.at[pl.ds(tile_base + b * _BUFW, _BUFW)],
                souts[cur])
        h_out[0].wait()
        h_out[1].wait()

    return k(x_flat, pos, nb)


def kernel(quantized):
    B, C, T = quantized.shape
    x = quantized.reshape(-1)
    out = _sc_jitter(x, jnp.asarray(_POS_BLK), jnp.asarray(_NB_BLK))
    return out.reshape(B, C, T)


# speedup vs baseline: 1.6683x; 1.5684x over previous
import functools

import jax
import jax.numpy as jnp
import numpy as np
from jax import lax
from jax.experimental import pallas as pl
from jax.experimental.pallas import tpu as pltpu
from jax.experimental.pallas import tpu_sc as plsc

_PROB = 0.12
_T = 4096
_ROWS = 16 * 256
_NW = 32
_ROWS_PER_TILE = _ROWS // _NW
_RB = 8
_BUFW = _RB * _T
_NBLK = _ROWS_PER_TILE // _RB
_L = 16


def _rotl(x, d):
    return ((x << np.uint32(d)) | (x >> np.uint32(32 - d))).astype(np.uint32)


def _threefry2x32_core(ks0, ks1, x0, x1):
    ks2 = np.uint32(ks0 ^ ks1 ^ np.uint32(0x1BD11BDA))
    rot = [[13, 15, 26, 6], [17, 29, 16, 24]]
    x0 = (x0 + ks0).astype(np.uint32)
    x1 = (x1 + ks1).astype(np.uint32)
    inject = [(ks1, ks2), (ks2, ks0), (ks0, ks1), (ks1, ks2), (ks2, ks0)]
    for g in range(5):
        for r in rot[g % 2]:
            x0 = (x0 + x1).astype(np.uint32)
            x1 = _rotl(x1, r)
            x1 = (x1 ^ x0).astype(np.uint32)
        a, b = inject[g]
        x0 = (x0 + a).astype(np.uint32)
        x1 = (x1 + b + np.uint32(g + 1)).astype(np.uint32)
    return x0, x1


def _uniform01(ks0, ks1, n):
    b1, b2 = _threefry2x32_core(
        ks0, ks1, np.zeros(n, dtype=np.uint32), np.arange(n, dtype=np.uint32))
    bits = (b1 ^ b2).astype(np.uint32)
    return ((bits >> np.uint32(9)) | np.uint32(0x3F800000)).view(np.float32) - np.float32(1.0)


def _jitter_index_constants():
    b1, b2 = _threefry2x32_core(np.uint32(0), np.uint32(42),
                                np.zeros(2, dtype=np.uint32),
                                np.arange(2, dtype=np.uint32))
    replace = _uniform01(b1[0], b2[0], _T) < np.float32(_PROB)
    direction = np.where(_uniform01(b1[1], b2[1], _T) < np.float32(0.5), -1, 1)
    i = np.arange(_T)
    offset = np.where(i == 0, 1, np.where(i == _T - 1, -1, direction))
    final = np.where(replace, i + offset, i)
    pos = np.nonzero(final != i)[0]
    nb = final[pos]
    safe = int(np.nonzero(final == i)[0][0])
    npad = (-len(pos)) % _L
    pos = np.concatenate([pos, np.full(npad, safe)]).astype(np.int32)
    nb = np.concatenate([nb, np.full(npad, safe)]).astype(np.int32)
    return pos, nb


_POS, _NB = _jitter_index_constants()
_NFIX = len(_POS)
_ROW_BLK = np.repeat(np.arange(_RB, dtype=np.int32), _NFIX)
_POS_BLK = np.tile(_POS, _RB)
_NB_BLK = np.tile(_NB, _RB)
_NFIXB = _POS_BLK.size
_NCHB = _NFIXB // _L


def _sc_jitter(x2d, row, pos, nb):
    mesh = plsc.VectorSubcoreMesh(core_axis_name="c", subcore_axis_name="s")

    @functools.partial(
        pl.kernel,
        mesh=mesh,
        out_type=jax.ShapeDtypeStruct((_ROWS, _T), jnp.float32),
        compiler_params=pltpu.CompilerParams(needs_layout_passes=False),
        scratch_types=[
            pltpu.VMEM((_NFIXB,), jnp.int32),
            pltpu.VMEM((_NFIXB,), jnp.int32),
            pltpu.VMEM((_NFIXB,), jnp.int32),
            pltpu.VMEM((_NFIXB,), jnp.float32),
            pltpu.VMEM((_RB, _T), jnp.float32),
            pltpu.VMEM((_RB, _T), jnp.float32),
            pltpu.SemaphoreType.DMA,
            pltpu.SemaphoreType.DMA,
            pltpu.SemaphoreType.DMA,
            pltpu.SemaphoreType.DMA,
        ],
    )
    def k(x_hbm, row_hbm, pos_hbm, nb_hbm, out_hbm, row_v, pos_v, nb_v, gat_v,
          buf0, buf1, si0, si1, so0, so1):
        wid = lax.axis_index("s") * 2 + lax.axis_index("c")
        pltpu.sync_copy(row_hbm, row_v)
        pltpu.sync_copy(pos_hbm, pos_v)
        pltpu.sync_copy(nb_hbm, nb_v)
        tile_row = wid * _ROWS_PER_TILE
        bufs = (buf0, buf1)
        sins = (si0, si1)
        souts = (so0, so1)

        def fixup(buf):
            def gather_body(j, c):
                sl = pl.ds(j * _L, _L)
                gat_v[sl] = plsc.load_gather(buf, [row_v[sl], nb_v[sl]])
                return c

            lax.fori_loop(0, _NCHB, gather_body, 0, unroll=8)

            def scatter_body(j, c):
                sl = pl.ds(j * _L, _L)
                plsc.store_scatter(buf, [row_v[sl], pos_v[sl]], gat_v[sl])
                return c

            lax.fori_loop(0, _NCHB, scatter_body, 0, unroll=8)

        h_in = [None, None]
        h_out = [None, None]
        h_in[0] = pltpu.async_copy(
            x_hbm.at[pl.ds(tile_row, _RB), :], bufs[0], sins[0])
        for b in range(_NBLK):
            cur = b % 2
            nxt = (b + 1) % 2
            if b + 1 < _NBLK:
                if h_out[nxt] is not None:
                    h_out[nxt].wait()
                h_in[nxt] = pltpu.async_copy(
                    x_hbm.at[pl.ds(tile_row + (b + 1) * _RB, _RB), :],
                    bufs[nxt], sins[nxt])
            h_in[cur].wait()
            fixup(bufs[cur])
            h_out[cur] = pltpu.async_copy(
                bufs[cur], out_hbm.at[pl.ds(tile_row + b * _RB, _RB), :],
                souts[cur])
        h_out[0].wait()
        h_out[1].wait()

    return k(x2d, row, pos, nb)


def kernel(quantized):
    B, C, T = quantized.shape
    x2d = quantized.reshape(B * C, T)
    out = _sc_jitter(x2d, jnp.asarray(_ROW_BLK), jnp.asarray(_POS_BLK),
                     jnp.asarray(_NB_BLK))
    return out.reshape(B, C, T)
